# baseline (speedup 1.0000x reference)
import functools

import jax
import jax.numpy as jnp
from jax import lax
from jax.experimental import pallas as pl
from jax.experimental.pallas import tpu as pltpu
from jax.experimental.pallas import tpu_sc as plsc

_POSE_NUM = 100000
_EMBED_DIM = 6
_BATCH = 16384

_NUM_CORES = 2
_NUM_SUBCORES = 16
_NUM_WORKERS = _NUM_CORES * _NUM_SUBCORES
_B_PER_W = _BATCH // _NUM_WORKERS
_HALF = _B_PER_W // 2

_mesh = plsc.VectorSubcoreMesh(core_axis_name="c", subcore_axis_name="s")


@functools.partial(
    pl.kernel,
    mesh=_mesh,
    out_type=jax.ShapeDtypeStruct((_EMBED_DIM, _BATCH), jnp.float32),
    scratch_types=[
        pltpu.VMEM((2, _HALF), jnp.int32),
        pltpu.VMEM((_EMBED_DIM, _B_PER_W), jnp.float32),
        pltpu.SemaphoreType.DMA,
        pltpu.SemaphoreType.DMA,
        pltpu.SemaphoreType.DMA,
    ],
    compiler_params=pltpu.CompilerParams(use_tc_tiling_on_sc=False),
)
def _gather_kernel(idx_hbm, table_hbm, out_hbm, idx_v, cols_v, isem, gsem, osem):
    wid = lax.axis_index("s") * _NUM_CORES + lax.axis_index("c")
    base = wid * _B_PER_W
    idx_loads = [
        pltpu.async_copy(
            idx_hbm.at[pl.ds(base + h * _HALF, _HALF)], idx_v.at[h], isem
        )
        for h in range(2)
    ]
    gathers = [None, None]
    writes = []
    for h in range(2):
        idx_loads[h].wait()
        gathers[h] = [
            pltpu.async_copy(
                table_hbm.at[j].at[idx_v.at[h]],
                cols_v.at[j, pl.ds(h * _HALF, _HALF)],
                gsem,
            )
            for j in range(_EMBED_DIM)
        ]
        if h == 1:
            for j in range(_EMBED_DIM):
                gathers[0][j].wait()
                writes.append(
                    pltpu.async_copy(
                        cols_v.at[j, pl.ds(0, _HALF)],
                        out_hbm.at[j, pl.ds(base, _HALF)],
                        osem,
                    )
                )
    for j in range(_EMBED_DIM):
        gathers[1][j].wait()
        writes.append(
            pltpu.async_copy(
                cols_v.at[j, pl.ds(_HALF, _HALF)],
                out_hbm.at[j, pl.ds(base + _HALF, _HALF)],
                osem,
            )
        )
    for w in writes:
        w.wait()


def kernel(indices, table):
    out_t = _gather_kernel(indices.astype(jnp.int32), table.T)
    return out_t.T


# speedup vs baseline: 1.0138x; 1.0138x over previous
import functools

import jax
import jax.numpy as jnp
from jax import lax
from jax.experimental import pallas as pl
from jax.experimental.pallas import tpu as pltpu
from jax.experimental.pallas import tpu_sc as plsc

_POSE_NUM = 100000
_EMBED_DIM = 6
_BATCH = 16384

_NUM_CORES = 2
_NUM_SUBCORES = 16
_NUM_WORKERS = _NUM_CORES * _NUM_SUBCORES
_B_PER_W = _BATCH // _NUM_WORKERS

_mesh = plsc.VectorSubcoreMesh(core_axis_name="c", subcore_axis_name="s")


@functools.partial(
    pl.kernel,
    mesh=_mesh,
    out_type=jax.ShapeDtypeStruct((_EMBED_DIM, _BATCH), jnp.float32),
    scratch_types=[
        pltpu.VMEM((_B_PER_W,), jnp.int32),
        pltpu.VMEM((_EMBED_DIM, _B_PER_W), jnp.float32),
        pltpu.SemaphoreType.DMA,
        pltpu.SemaphoreType.DMA,
    ],
    compiler_params=pltpu.CompilerParams(use_tc_tiling_on_sc=False),
)
def _gather_kernel(idx_hbm, table_hbm, out_hbm, idx_v, cols_v, gsem, osem):
    wid = lax.axis_index("s") * _NUM_CORES + lax.axis_index("c")
    base = wid * _B_PER_W
    pltpu.sync_copy(idx_hbm.at[pl.ds(base, _B_PER_W)], idx_v)
    gathers = [
        pltpu.async_copy(table_hbm.at[j].at[idx_v], cols_v.at[j], gsem)
        for j in range(_EMBED_DIM)
    ]
    writes = []
    for j in range(_EMBED_DIM):
        gathers[j].wait()
        writes.append(
            pltpu.async_copy(
                cols_v.at[j], out_hbm.at[j, pl.ds(base, _B_PER_W)], osem
            )
        )
    for w in writes:
        w.wait()


def kernel(indices, table):
    out_t = _gather_kernel(indices.astype(jnp.int32), table.T)
    return out_t.T
